# baseline (speedup 1.0000x reference)
import jax
import jax.numpy as jnp
from jax.experimental import pallas as pl
from jax.experimental.pallas import tpu as pltpu

N = 10000
D = 128
BM = 512
NJ = 10


def _gcn_kernel(x_ref, w_ref, adj_ref, bias_ref, out_ref, sup_ref):
    j = pl.program_id(1)

    @pl.when(j == 0)
    def _():
        sup = jnp.dot(x_ref[...], w_ref[...],
                      preferred_element_type=jnp.float32)
        sup_ref[...] = sup.astype(jnp.bfloat16)

    acc = jnp.dot(adj_ref[...].astype(jnp.bfloat16), sup_ref[...],
                  preferred_element_type=jnp.float32)
    out_ref[...] = acc + bias_ref[...]


@jax.jit
def kernel(input, adj, weight, bias):
    bias2d = bias.reshape(1, D)
    return pl.pallas_call(
        _gcn_kernel,
        grid=(2, NJ),
        in_specs=[
            pl.BlockSpec((N, D), lambda i, j: (0, 0)),
            pl.BlockSpec((D, D), lambda i, j: (0, 0)),
            pl.BlockSpec((BM, N), lambda i, j: (i * NJ + j, 0)),
            pl.BlockSpec((1, D), lambda i, j: (0, 0)),
        ],
        out_specs=pl.BlockSpec((BM, D), lambda i, j: (i * NJ + j, 0)),
        out_shape=jax.ShapeDtypeStruct((N, D), jnp.float32),
        scratch_shapes=[pltpu.VMEM((N, D), jnp.bfloat16)],
        compiler_params=pltpu.CompilerParams(
            dimension_semantics=("parallel", "arbitrary"),
        ),
    )(input, weight, adj, bias2d)


# speedup vs baseline: 1.0162x; 1.0162x over previous
import jax
import jax.numpy as jnp
from jax.experimental import pallas as pl
from jax.experimental.pallas import tpu as pltpu

N = 10000
D = 128
BM = 392
NJ = 13


def _gcn_kernel(x_ref, w_ref, adj_ref, bias_ref, out_ref, sup_ref):
    j = pl.program_id(1)

    @pl.when(j == 0)
    def _():
        sup = jnp.dot(x_ref[...], w_ref[...],
                      preferred_element_type=jnp.float32)
        sup_ref[...] = sup.astype(jnp.bfloat16)

    acc = jnp.dot(adj_ref[...].astype(jnp.bfloat16), sup_ref[...],
                  preferred_element_type=jnp.float32)
    out_ref[...] = acc + bias_ref[...]


@jax.jit
def kernel(input, adj, weight, bias):
    bias2d = bias.reshape(1, D)
    return pl.pallas_call(
        _gcn_kernel,
        grid=(2, NJ),
        in_specs=[
            pl.BlockSpec((N, D), lambda i, j: (0, 0)),
            pl.BlockSpec((D, D), lambda i, j: (0, 0)),
            pl.BlockSpec((BM, N), lambda i, j: (i * NJ + j, 0)),
            pl.BlockSpec((1, D), lambda i, j: (0, 0)),
        ],
        out_specs=pl.BlockSpec((BM, D), lambda i, j: (i * NJ + j, 0)),
        out_shape=jax.ShapeDtypeStruct((N, D), jnp.float32),
        scratch_shapes=[pltpu.VMEM((N, D), jnp.bfloat16)],
        compiler_params=pltpu.CompilerParams(
            dimension_semantics=("parallel", "arbitrary"),
        ),
    )(input, weight, adj, bias2d)


# speedup vs baseline: 1.0249x; 1.0085x over previous
import jax
import jax.numpy as jnp
from jax.experimental import pallas as pl
from jax.experimental.pallas import tpu as pltpu

N = 10000
D = 128
BM = 360
NJ = 14


def _gcn_kernel(x_ref, w_ref, adj_ref, bias_ref, out_ref, sup_ref):
    j = pl.program_id(1)

    @pl.when(j == 0)
    def _():
        sup = jnp.dot(x_ref[...], w_ref[...],
                      preferred_element_type=jnp.float32)
        sup_ref[...] = sup.astype(jnp.bfloat16)

    acc = jnp.dot(adj_ref[...].astype(jnp.bfloat16), sup_ref[...],
                  preferred_element_type=jnp.float32)
    out_ref[...] = acc + bias_ref[...]


@jax.jit
def kernel(input, adj, weight, bias):
    bias2d = bias.reshape(1, D)
    return pl.pallas_call(
        _gcn_kernel,
        grid=(2, NJ),
        in_specs=[
            pl.BlockSpec((N, D), lambda i, j: (0, 0)),
            pl.BlockSpec((D, D), lambda i, j: (0, 0)),
            pl.BlockSpec((BM, N), lambda i, j: (i * NJ + j, 0)),
            pl.BlockSpec((1, D), lambda i, j: (0, 0)),
        ],
        out_specs=pl.BlockSpec((BM, D), lambda i, j: (i * NJ + j, 0)),
        out_shape=jax.ShapeDtypeStruct((N, D), jnp.float32),
        scratch_shapes=[pltpu.VMEM((N, D), jnp.bfloat16)],
        compiler_params=pltpu.CompilerParams(
            dimension_semantics=("parallel", "arbitrary"),
        ),
    )(input, weight, adj, bias2d)


# speedup vs baseline: 1.0254x; 1.0006x over previous
import jax
import jax.numpy as jnp
from jax.experimental import pallas as pl
from jax.experimental.pallas import tpu as pltpu

N = 10000
D = 128
BM = 336
NJ = 15


def _gcn_kernel(x_ref, w_ref, adj_ref, bias_ref, out_ref, sup_ref):
    j = pl.program_id(1)

    @pl.when(j == 0)
    def _():
        sup = jnp.dot(x_ref[...], w_ref[...],
                      preferred_element_type=jnp.float32)
        sup_ref[...] = sup.astype(jnp.bfloat16)

    acc = jnp.dot(adj_ref[...].astype(jnp.bfloat16), sup_ref[...],
                  preferred_element_type=jnp.float32)
    out_ref[...] = acc + bias_ref[...]


@jax.jit
def kernel(input, adj, weight, bias):
    bias2d = bias.reshape(1, D)
    return pl.pallas_call(
        _gcn_kernel,
        grid=(2, NJ),
        in_specs=[
            pl.BlockSpec((N, D), lambda i, j: (0, 0)),
            pl.BlockSpec((D, D), lambda i, j: (0, 0)),
            pl.BlockSpec((BM, N), lambda i, j: (i * NJ + j, 0)),
            pl.BlockSpec((1, D), lambda i, j: (0, 0)),
        ],
        out_specs=pl.BlockSpec((BM, D), lambda i, j: (i * NJ + j, 0)),
        out_shape=jax.ShapeDtypeStruct((N, D), jnp.float32),
        scratch_shapes=[pltpu.VMEM((N, D), jnp.bfloat16)],
        compiler_params=pltpu.CompilerParams(
            dimension_semantics=("parallel", "arbitrary"),
        ),
    )(input, weight, adj, bias2d)


# speedup vs baseline: 1.0288x; 1.0032x over previous
import jax
import jax.numpy as jnp
from jax.experimental import pallas as pl
from jax.experimental.pallas import tpu as pltpu

N = 10000
D = 128
BM = 296
NJ = 17


def _gcn_kernel(x_ref, w_ref, adj_ref, bias_ref, out_ref, sup_ref):
    j = pl.program_id(1)

    @pl.when(j == 0)
    def _():
        sup = jnp.dot(x_ref[...], w_ref[...],
                      preferred_element_type=jnp.float32)
        sup_ref[...] = sup.astype(jnp.bfloat16)

    acc = jnp.dot(adj_ref[...].astype(jnp.bfloat16), sup_ref[...],
                  preferred_element_type=jnp.float32)
    out_ref[...] = acc + bias_ref[...]


@jax.jit
def kernel(input, adj, weight, bias):
    bias2d = bias.reshape(1, D)
    return pl.pallas_call(
        _gcn_kernel,
        grid=(2, NJ),
        in_specs=[
            pl.BlockSpec((N, D), lambda i, j: (0, 0)),
            pl.BlockSpec((D, D), lambda i, j: (0, 0)),
            pl.BlockSpec((BM, N), lambda i, j: (i * NJ + j, 0)),
            pl.BlockSpec((1, D), lambda i, j: (0, 0)),
        ],
        out_specs=pl.BlockSpec((BM, D), lambda i, j: (i * NJ + j, 0)),
        out_shape=jax.ShapeDtypeStruct((N, D), jnp.float32),
        scratch_shapes=[pltpu.VMEM((N, D), jnp.bfloat16)],
        compiler_params=pltpu.CompilerParams(
            dimension_semantics=("parallel", "arbitrary"),
        ),
    )(input, weight, adj, bias2d)
